# baseline (speedup 1.0000x reference)
import dataclasses
import functools

import jax
import jax.numpy as jnp
from jax import lax
from jax.experimental import pallas as pl
from jax.experimental.pallas import tpu as pltpu
from jax.experimental.pallas import tpu_sc as plsc


@functools.lru_cache(maxsize=None)
def _fle_kernel(B, L, D):
    info = plsc.get_sparse_core_info()
    NC, NS, LN = info.num_cores, info.num_subcores, info.num_lanes
    NW = NC * NS
    C = 32
    b_per_w = B // NW
    n_chunks = b_per_w // C
    nfull = L // LN
    rem = L - nfull * LN

    mesh = plsc.VectorSubcoreMesh(core_axis_name="c", subcore_axis_name="s")
    cp = pltpu.CompilerParams()
    if "needs_layout_passes" in pltpu.CompilerParams.__dataclass_fields__:
        cp = dataclasses.replace(cp, needs_layout_passes=False)

    @functools.partial(
        pl.kernel,
        out_type=jax.ShapeDtypeStruct((B, 2 * D), jnp.float32),
        mesh=mesh,
        compiler_params=cp,
        scratch_types=[
            pltpu.VMEM((C, L), jnp.int32),
            pltpu.VMEM((C, 8, D), jnp.float32),
            pltpu.VMEM((C, 8, D), jnp.float32),
            pltpu.VMEM((C, 2 * D), jnp.float32),
            pltpu.SemaphoreType.DMA,
            pltpu.SemaphoreType.DMA,
            pltpu.SemaphoreType.DMA,
        ],
    )
    def k(mask_hbm, tok4_hbm, out_hbm, mask_v, firstg_v, lastg_v, comb_v,
          semm, semf, seml):
        wid = lax.axis_index("s") * NC + lax.axis_index("c")
        lanes = lax.iota(jnp.int32, LN)
        tail_keep = (lanes >= LN - rem).astype(jnp.int32)

        @pl.loop(0, n_chunks)
        def _(ch):
            row0 = wid * b_per_w + ch * C
            cpm = pltpu.async_copy(mask_hbm.at[pl.ds(row0, C), :], mask_v,
                                   semm)
            cpf = pltpu.async_copy(tok4_hbm.at[pl.ds(row0, C), 0], firstg_v,
                                   semf)
            cpm.wait()
            copies = []
            js_list = []
            for r in range(C):
                acc = mask_v[r, pl.ds(nfull * LN - LN + rem, LN)] * tail_keep
                for cblk in range(nfull):
                    acc = acc + mask_v[r, pl.ds(cblk * LN, LN)]
                ln = jnp.sum(acc)
                adj = jnp.maximum(ln - 1, 0)
                jg = adj >> 3
                copies.append(
                    pltpu.async_copy(tok4_hbm.at[row0 + r, jg],
                                     lastg_v.at[r], seml))
                js_list.append(adj & 7)
            cpf.wait()
            for r in range(C):
                for c in range(0, D, LN):
                    comb_v[r, pl.ds(c, LN)] = firstg_v[r, 0, pl.ds(c, LN)]
            for cp_ in copies:
                cp_.wait()
            for r in range(C):
                js = js_list[r]
                for c in range(0, D, LN):
                    comb_v[r, pl.ds(D + c, LN)] = lastg_v[r, js, pl.ds(c, LN)]
            pltpu.sync_copy(comb_v, out_hbm.at[pl.ds(row0, C), :])

    return k


def kernel(mask, embedded_tokens):
    B, L, D = embedded_tokens.shape
    tok4 = embedded_tokens.reshape(B, L // 8, 8, D)
    return _fle_kernel(B, L, D)(mask.astype(jnp.int32), tok4)
